# baseline (device time: 31679 ns/iter reference)
import jax
import jax.numpy as jnp
from jax import lax
from jax.experimental import pallas as pl
from jax.experimental.pallas import tpu as pltpu

N_DEV = 16
B, SQ, D = 2, 128, 512
HQ_LOCAL, DH = 8, 64
ROWS = B * SQ
QROWS = 64

QFLIP = {1: 2, 2: 1, 3: 3}


def kernel(x, Wq, Wo, K_ext, V_ext):
    d_model = Wo.shape[1]

    def body(x_ref, wq_ref, wo_ref, k_ref, v_ref, out_ref,
             recv_ref, send_buf_ref, send_sems, recv_sems):
        my = lax.axis_index("i")
        bit0 = my & 1
        bit1 = (my >> 1) & 1
        q_me = 2 * bit0 + bit1
        s = q_me * QROWS

        barrier_sem = pltpu.get_barrier_semaphore()
        for xr in (1, 2, 3, 4, 8):
            pl.semaphore_signal(barrier_sem, inc=1,
                                device_id=(my ^ xr,),
                                device_id_type=pl.DeviceIdType.MESH)

        a_rdma = {}
        for XR in (1, 2, 3):
            qp = q_me ^ QFLIP[XR]
            a_rdma[XR] = pltpu.make_async_remote_copy(
                src_ref=send_buf_ref.at[pl.ds(qp * QROWS, QROWS)],
                dst_ref=recv_ref.at[pl.ds((XR - 1) * QROWS, QROWS)],
                send_sem=send_sems.at[XR - 1],
                recv_sem=recv_sems.at[XR - 1],
                device_id=(my ^ XR,),
                device_id_type=pl.DeviceIdType.MESH,
            )

        def compute_quarter(q):
            b, r0 = q // 2, (q % 2) * QROWS
            xq = x_ref[b][r0:r0 + QROWS, :]
            qq = jnp.dot(xq, wq_ref[...],
                         preferred_element_type=jnp.float32)
            q3 = qq.reshape(QROWS, HQ_LOCAL, DH)
            kv = k_ref[b]
            vv = v_ref[b]
            outs_h = []
            for h in range(HQ_LOCAL):
                qh = q3[:, h, :]
                kh = kv[:, h, :]
                sc = jnp.dot(qh, kh.T,
                             preferred_element_type=jnp.float32) * 0.125
                sc = sc - jnp.max(sc, axis=-1, keepdims=True)
                p = jnp.exp(sc)
                p = p / jnp.sum(p, axis=-1, keepdims=True)
                outs_h.append(jnp.dot(p, vv[:, h, :],
                                      preferred_element_type=jnp.float32))
            attn_q = jnp.concatenate(outs_h, axis=-1)
            out_ref[pl.ds(q * QROWS, QROWS)] = jnp.dot(
                attn_q, wo_ref[...], preferred_element_type=jnp.float32)
            send_buf_ref[pl.ds(q * QROWS, QROWS)] = out_ref[
                pl.ds(q * QROWS, QROWS)].astype(jnp.bfloat16)

        compute_quarter(0)
        pl.semaphore_wait(barrier_sem, 5)
        for q in range(4):
            if q > 0:
                compute_quarter(q)
            for XR in (1, 2, 3):
                @pl.when((q_me ^ QFLIP[XR]) == q)
                def _(XR=XR):
                    a_rdma[XR].start()

        for XR in (1, 2, 3):
            a_rdma[XR].wait_recv()
        out_ref[pl.ds(s, QROWS)] = (
            out_ref[pl.ds(s, QROWS)]
            + recv_ref[pl.ds(0, QROWS)].astype(jnp.float32)
            + recv_ref[pl.ds(QROWS, QROWS)].astype(jnp.float32)
            + recv_ref[pl.ds(2 * QROWS, QROWS)].astype(jnp.float32)
        )
        for XR in (1, 2, 3):
            a_rdma[XR].wait_send()

        for i, XR in enumerate((4, 8)):
            send_buf_ref[pl.ds(s, QROWS)] = out_ref[
                pl.ds(s, QROWS)].astype(jnp.bfloat16)
            off = (3 + i) * QROWS
            rdma = pltpu.make_async_remote_copy(
                src_ref=send_buf_ref.at[pl.ds(s, QROWS)],
                dst_ref=recv_ref.at[pl.ds(off, QROWS)],
                send_sem=send_sems.at[3 + i],
                recv_sem=recv_sems.at[3 + i],
                device_id=(my ^ XR,),
                device_id_type=pl.DeviceIdType.MESH,
            )
            rdma.start()
            rdma.wait()
            out_ref[pl.ds(s, QROWS)] = (
                out_ref[pl.ds(s, QROWS)]
                + recv_ref[pl.ds(off, QROWS)].astype(jnp.float32)
            )

        send_buf_ref[pl.ds(s, QROWS)] = out_ref[
            pl.ds(s, QROWS)].astype(jnp.bfloat16)
        c_rdma = {}
        for XR in (1, 2, 3):
            c_rdma[XR] = pltpu.make_async_remote_copy(
                src_ref=send_buf_ref.at[pl.ds(s, QROWS)],
                dst_ref=recv_ref.at[pl.ds((5 + XR - 1) * QROWS, QROWS)],
                send_sem=send_sems.at[5 + XR - 1],
                recv_sem=recv_sems.at[5 + XR - 1],
                device_id=(my ^ XR,),
                device_id_type=pl.DeviceIdType.MESH,
            )
            c_rdma[XR].start()
        for XR in (1, 2, 3):
            c_rdma[XR].wait_recv()
            qp = q_me ^ QFLIP[XR]
            out_ref[pl.ds(qp * QROWS, QROWS)] = recv_ref[
                pl.ds((5 + XR - 1) * QROWS, QROWS)].astype(jnp.float32)
        for XR in (1, 2, 3):
            c_rdma[XR].wait_send()

    out = pl.pallas_call(
        body,
        out_shape=jax.ShapeDtypeStruct((ROWS, d_model), jnp.float32),
        in_specs=[pl.BlockSpec(memory_space=pltpu.VMEM)] * 5,
        out_specs=pl.BlockSpec(memory_space=pltpu.VMEM),
        scratch_shapes=[
            pltpu.VMEM((512, d_model), jnp.bfloat16),
            pltpu.VMEM((ROWS, d_model), jnp.bfloat16),
            pltpu.SemaphoreType.DMA((8,)),
            pltpu.SemaphoreType.DMA((8,)),
        ],
        compiler_params=pltpu.CompilerParams(collective_id=0),
    )(x, Wq, Wo, K_ext, V_ext)
    return out.reshape(B, SQ, d_model)


# device time: 25599 ns/iter; 1.2375x vs baseline; 1.2375x over previous
import jax
import jax.numpy as jnp
from jax import lax
from jax.experimental import pallas as pl
from jax.experimental.pallas import tpu as pltpu

N_DEV = 16
B, SQ, D = 2, 128, 512
HQ_LOCAL, DH = 8, 64
ROWS = B * SQ
HALF = ROWS // 2


def kernel(x, Wq, Wo, K_ext, V_ext):
    d_model = Wo.shape[1]

    ROUNDS = [
        ("rs", 2, 64, 128),
        ("bf", 4, 64, 192),
        ("bf", 8, 64, 256),
    ]

    def body(x_ref, wq_ref, wo_ref, k_ref, v_ref, out_ref,
             recv_ref, send_buf_ref, send_sems, recv_sems):
        my = lax.axis_index("i")
        bit0 = my & 1

        barrier_sem = pltpu.get_barrier_semaphore()
        for xr in (1, 2, 3, 4, 8):
            pl.semaphore_signal(barrier_sem, inc=1,
                                device_id=(my ^ xr,),
                                device_id_type=pl.DeviceIdType.MESH)

        def compute_half(b):
            xb = x_ref[b]
            qb = jnp.dot(xb, wq_ref[...],
                         preferred_element_type=jnp.float32)
            q3 = qb.reshape(SQ, HQ_LOCAL, DH)
            kv = k_ref[b]
            vv = v_ref[b]
            outs_h = []
            for h in range(HQ_LOCAL):
                qh = q3[:, h, :]
                kh = kv[:, h, :]
                sc = jnp.dot(qh, kh.T,
                             preferred_element_type=jnp.float32) * 0.125
                sc = sc - jnp.max(sc, axis=-1, keepdims=True)
                p = jnp.exp(sc)
                p = p / jnp.sum(p, axis=-1, keepdims=True)
                outs_h.append(jnp.dot(p, vv[:, h, :],
                                      preferred_element_type=jnp.float32))
            attn_b = jnp.concatenate(outs_h, axis=-1)
            out_ref[pl.ds(b * HALF, HALF)] = jnp.dot(
                attn_b, wo_ref[...], preferred_element_type=jnp.float32)

        @pl.when(bit0 == 0)
        def _():
            compute_half(1)

        @pl.when(bit0 == 1)
        def _():
            compute_half(0)

        pl.semaphore_wait(barrier_sem, 5)

        send_start0 = (1 - bit0) * HALF
        send_buf_ref[pl.ds(0, HALF)] = out_ref[
            pl.ds(send_start0, HALF)].astype(jnp.bfloat16)
        rdma0 = pltpu.make_async_remote_copy(
            src_ref=send_buf_ref.at[pl.ds(0, HALF)],
            dst_ref=recv_ref.at[pl.ds(0, HALF)],
            send_sem=send_sems.at[0],
            recv_sem=recv_sems.at[0],
            device_id=(my ^ 1,),
            device_id_type=pl.DeviceIdType.MESH,
        )
        rdma0.start()

        @pl.when(bit0 == 0)
        def _():
            compute_half(0)

        @pl.when(bit0 == 1)
        def _():
            compute_half(1)

        rdma0.wait()
        s = bit0 * HALF
        out_ref[pl.ds(s, HALF)] = (
            out_ref[pl.ds(s, HALF)]
            + recv_ref[pl.ds(0, HALF)].astype(jnp.float32)
        )

        for idx, (kind, xr, L, off) in enumerate(ROUNDS, start=1):
            partner = my ^ xr
            bit = (my & xr) // xr
            if kind == "rs":
                src_start = s + (1 - bit) * L
            else:
                src_start = s
            send_buf_ref[pl.ds(0, L)] = out_ref[
                pl.ds(src_start, L)].astype(jnp.bfloat16)
            rdma = pltpu.make_async_remote_copy(
                src_ref=send_buf_ref.at[pl.ds(0, L)],
                dst_ref=recv_ref.at[pl.ds(off, L)],
                send_sem=send_sems.at[idx],
                recv_sem=recv_sems.at[idx],
                device_id=(partner,),
                device_id_type=pl.DeviceIdType.MESH,
            )
            rdma.start()
            rdma.wait()
            if kind == "rs":
                s = s + bit * L
            out_ref[pl.ds(s, L)] = (
                out_ref[pl.ds(s, L)]
                + recv_ref[pl.ds(off, L)].astype(jnp.float32)
            )

        send_buf_ref[pl.ds(0, 64)] = out_ref[pl.ds(s, 64)].astype(jnp.bfloat16)
        c_rdma = []
        for i, XR in enumerate((1, 2, 3)):
            c = pltpu.make_async_remote_copy(
                src_ref=send_buf_ref.at[pl.ds(0, 64)],
                dst_ref=recv_ref.at[pl.ds(320 + i * 64, 64)],
                send_sem=send_sems.at[4 + i],
                recv_sem=recv_sems.at[4 + i],
                device_id=(my ^ XR,),
                device_id_type=pl.DeviceIdType.MESH,
            )
            c.start()
            c_rdma.append(c)
        for i, XR in enumerate((1, 2, 3)):
            c_rdma[i].wait_recv()
            p = my ^ XR
            sp = (p & 1) * HALF + ((p >> 1) & 1) * 64
            out_ref[pl.ds(sp, 64)] = recv_ref[
                pl.ds(320 + i * 64, 64)].astype(jnp.float32)
        for c in c_rdma:
            c.wait_send()

    out = pl.pallas_call(
        body,
        out_shape=jax.ShapeDtypeStruct((ROWS, d_model), jnp.float32),
        in_specs=[pl.BlockSpec(memory_space=pltpu.VMEM)] * 5,
        out_specs=pl.BlockSpec(memory_space=pltpu.VMEM),
        scratch_shapes=[
            pltpu.VMEM((512, d_model), jnp.bfloat16),
            pltpu.VMEM((HALF, d_model), jnp.bfloat16),
            pltpu.SemaphoreType.DMA((7,)),
            pltpu.SemaphoreType.DMA((7,)),
        ],
        compiler_params=pltpu.CompilerParams(collective_id=0),
    )(x, Wq, Wo, K_ext, V_ext)
    return out.reshape(B, SQ, d_model)


# device time: 24684 ns/iter; 1.2834x vs baseline; 1.0371x over previous
import jax
import jax.numpy as jnp
from jax import lax
from jax.experimental import pallas as pl
from jax.experimental.pallas import tpu as pltpu

N_DEV = 16
B, SQ, D = 2, 128, 512
HQ_LOCAL, DH = 8, 64
ROWS = B * SQ
HALF = ROWS // 2


def kernel(x, Wq, Wo, K_ext, V_ext):
    d_model = Wo.shape[1]

    def body(x_ref, wq_ref, wo_ref, k_ref, v_ref, out_ref,
             recv_ref, send_buf_ref, send_sems, recv_sems):
        my = lax.axis_index("i")
        bit0 = my & 1
        bf16 = jnp.bfloat16

        barrier_sem = pltpu.get_barrier_semaphore()
        for xr in (1, 2, 3, 4, 8, 12):
            pl.semaphore_signal(barrier_sem, inc=1,
                                device_id=(my ^ xr,),
                                device_id_type=pl.DeviceIdType.MESH)

        def compute_half(b):
            xb = x_ref[b]
            qb = jnp.dot(xb.astype(bf16), wq_ref[...].astype(bf16),
                         preferred_element_type=jnp.float32)
            q3 = qb.reshape(SQ, HQ_LOCAL, DH)
            kv = k_ref[b]
            vv = v_ref[b]
            outs_h = []
            for h in range(HQ_LOCAL):
                qh = q3[:, h, :].astype(bf16)
                kh = kv[:, h, :].astype(bf16)
                sc = jnp.dot(qh, kh.T,
                             preferred_element_type=jnp.float32) * 0.125
                sc = sc - jnp.max(sc, axis=-1, keepdims=True)
                p = jnp.exp(sc)
                p = p / jnp.sum(p, axis=-1, keepdims=True)
                outs_h.append(jnp.dot(p.astype(bf16), vv[:, h, :].astype(bf16),
                                      preferred_element_type=jnp.float32))
            attn_b = jnp.concatenate(outs_h, axis=-1)
            out_ref[pl.ds(b * HALF, HALF)] = jnp.dot(
                attn_b.astype(bf16), wo_ref[...].astype(bf16),
                preferred_element_type=jnp.float32)

        @pl.when(bit0 == 0)
        def _():
            compute_half(1)

        @pl.when(bit0 == 1)
        def _():
            compute_half(0)

        pl.semaphore_wait(barrier_sem, 5)

        send_start0 = (1 - bit0) * HALF
        send_buf_ref[pl.ds(0, HALF)] = out_ref[
            pl.ds(send_start0, HALF)].astype(jnp.bfloat16)
        rdma0 = pltpu.make_async_remote_copy(
            src_ref=send_buf_ref.at[pl.ds(0, HALF)],
            dst_ref=recv_ref.at[pl.ds(0, HALF)],
            send_sem=send_sems.at[0],
            recv_sem=recv_sems.at[0],
            device_id=(my ^ 1,),
            device_id_type=pl.DeviceIdType.MESH,
        )
        rdma0.start()

        @pl.when(bit0 == 0)
        def _():
            compute_half(0)

        @pl.when(bit0 == 1)
        def _():
            compute_half(1)

        rdma0.wait()
        s = bit0 * HALF
        out_ref[pl.ds(s, HALF)] = (
            out_ref[pl.ds(s, HALF)]
            + recv_ref[pl.ds(0, HALF)].astype(jnp.float32)
        )

        bit1 = (my >> 1) & 1
        send_buf_ref[pl.ds(0, 64)] = out_ref[
            pl.ds(s + (1 - bit1) * 64, 64)].astype(jnp.bfloat16)
        rdma1 = pltpu.make_async_remote_copy(
            src_ref=send_buf_ref.at[pl.ds(0, 64)],
            dst_ref=recv_ref.at[pl.ds(128, 64)],
            send_sem=send_sems.at[1],
            recv_sem=recv_sems.at[1],
            device_id=(my ^ 2,),
            device_id_type=pl.DeviceIdType.MESH,
        )
        rdma1.start()
        rdma1.wait()
        s = s + bit1 * 64
        out_ref[pl.ds(s, 64)] = (
            out_ref[pl.ds(s, 64)]
            + recv_ref[pl.ds(128, 64)].astype(jnp.float32)
        )

        send_buf_ref[pl.ds(0, 64)] = out_ref[pl.ds(s, 64)].astype(jnp.bfloat16)
        z_rdma = []
        for i, XR in enumerate((4, 8, 12)):
            z = pltpu.make_async_remote_copy(
                src_ref=send_buf_ref.at[pl.ds(0, 64)],
                dst_ref=recv_ref.at[pl.ds(192 + i * 64, 64)],
                send_sem=send_sems.at[2 + i],
                recv_sem=recv_sems.at[2 + i],
                device_id=(my ^ XR,),
                device_id_type=pl.DeviceIdType.MESH,
            )
            z.start()
            z_rdma.append(z)
        for z in z_rdma:
            z.wait_recv()
        out_ref[pl.ds(s, 64)] = (
            out_ref[pl.ds(s, 64)]
            + recv_ref[pl.ds(192, 64)].astype(jnp.float32)
            + recv_ref[pl.ds(256, 64)].astype(jnp.float32)
            + recv_ref[pl.ds(320, 64)].astype(jnp.float32)
        )
        for z in z_rdma:
            z.wait_send()

        send_buf_ref[pl.ds(0, 64)] = out_ref[pl.ds(s, 64)].astype(jnp.bfloat16)
        c_rdma = []
        for i, XR in enumerate((1, 2, 3)):
            c = pltpu.make_async_remote_copy(
                src_ref=send_buf_ref.at[pl.ds(0, 64)],
                dst_ref=recv_ref.at[pl.ds(384 + i * 64, 64)],
                send_sem=send_sems.at[5 + i],
                recv_sem=recv_sems.at[5 + i],
                device_id=(my ^ XR,),
                device_id_type=pl.DeviceIdType.MESH,
            )
            c.start()
            c_rdma.append(c)
        for i, XR in enumerate((1, 2, 3)):
            c_rdma[i].wait_recv()
            p = my ^ XR
            sp = (p & 1) * HALF + ((p >> 1) & 1) * 64
            out_ref[pl.ds(sp, 64)] = recv_ref[
                pl.ds(384 + i * 64, 64)].astype(jnp.float32)
        for c in c_rdma:
            c.wait_send()

    out = pl.pallas_call(
        body,
        out_shape=jax.ShapeDtypeStruct((ROWS, d_model), jnp.float32),
        in_specs=[pl.BlockSpec(memory_space=pltpu.VMEM)] * 5,
        out_specs=pl.BlockSpec(memory_space=pltpu.VMEM),
        scratch_shapes=[
            pltpu.VMEM((576, d_model), jnp.bfloat16),
            pltpu.VMEM((HALF, d_model), jnp.bfloat16),
            pltpu.SemaphoreType.DMA((8,)),
            pltpu.SemaphoreType.DMA((8,)),
        ],
        compiler_params=pltpu.CompilerParams(collective_id=0),
    )(x, Wq, Wo, K_ext, V_ext)
    return out.reshape(B, SQ, d_model)


# device time: 24423 ns/iter; 1.2971x vs baseline; 1.0107x over previous
import jax
import jax.numpy as jnp
from jax import lax
from jax.experimental import pallas as pl
from jax.experimental.pallas import tpu as pltpu

N_DEV = 16
B, SQ, D = 2, 128, 512
HQ_LOCAL, DH = 8, 64
ROWS = B * SQ
HALF = ROWS // 2


def kernel(x, Wq, Wo, K_ext, V_ext):
    d_model = Wo.shape[1]

    def body(x_ref, wq_ref, wo_ref, k_ref, v_ref, out_ref,
             recv_ref, send_buf_ref, send_sems, recv_sems):
        my = lax.axis_index("i")
        bit0 = my & 1
        bit1 = (my >> 1) & 1
        bf16 = jnp.bfloat16

        barrier_sem = pltpu.get_barrier_semaphore()
        for xr in (1, 2, 3, 4, 8, 12):
            pl.semaphore_signal(barrier_sem, inc=1,
                                device_id=(my ^ xr,),
                                device_id_type=pl.DeviceIdType.MESH)

        def compute_half(b):
            xb = x_ref[b]
            qb = jnp.dot(xb.astype(bf16), wq_ref[...].astype(bf16),
                         preferred_element_type=jnp.float32)
            q3 = qb.reshape(SQ, HQ_LOCAL, DH)
            kv = k_ref[b]
            vv = v_ref[b]
            outs_h = []
            for h in range(HQ_LOCAL):
                qh = q3[:, h, :].astype(bf16)
                kh = kv[:, h, :].astype(bf16)
                sc = jnp.dot(qh, kh.T,
                             preferred_element_type=jnp.float32) * 0.125
                sc = sc - jnp.max(sc, axis=-1, keepdims=True)
                p = jnp.exp(sc)
                p = p / jnp.sum(p, axis=-1, keepdims=True)
                outs_h.append(jnp.dot(p.astype(bf16), vv[:, h, :].astype(bf16),
                                      preferred_element_type=jnp.float32))
            attn_b = jnp.concatenate(outs_h, axis=-1)
            out_ref[pl.ds(b * HALF, HALF)] = jnp.dot(
                attn_b.astype(bf16), wo_ref[...].astype(bf16),
                preferred_element_type=jnp.float32)

        @pl.when(bit1 == 0)
        def _():
            compute_half(1)

        @pl.when(bit1 == 1)
        def _():
            compute_half(0)

        pl.semaphore_wait(barrier_sem, 5)

        send_start0 = (1 - bit1) * HALF
        send_buf_ref[pl.ds(0, HALF)] = out_ref[
            pl.ds(send_start0, HALF)].astype(jnp.bfloat16)
        rdma0 = pltpu.make_async_remote_copy(
            src_ref=send_buf_ref.at[pl.ds(0, HALF)],
            dst_ref=recv_ref.at[pl.ds(0, HALF)],
            send_sem=send_sems.at[0],
            recv_sem=recv_sems.at[0],
            device_id=(my ^ 3,),
            device_id_type=pl.DeviceIdType.MESH,
        )
        rdma0.start()

        @pl.when(bit1 == 0)
        def _():
            compute_half(0)

        @pl.when(bit1 == 1)
        def _():
            compute_half(1)

        rdma0.wait()
        s = bit1 * HALF
        out_ref[pl.ds(s, HALF)] = (
            out_ref[pl.ds(s, HALF)]
            + recv_ref[pl.ds(0, HALF)].astype(jnp.float32)
        )

        send_buf_ref[pl.ds(0, 64)] = out_ref[
            pl.ds(s + (1 - bit0) * 64, 64)].astype(jnp.bfloat16)
        rdma1 = pltpu.make_async_remote_copy(
            src_ref=send_buf_ref.at[pl.ds(0, 64)],
            dst_ref=recv_ref.at[pl.ds(128, 64)],
            send_sem=send_sems.at[1],
            recv_sem=recv_sems.at[1],
            device_id=(my ^ 1,),
            device_id_type=pl.DeviceIdType.MESH,
        )
        rdma1.start()
        rdma1.wait()
        s = s + bit0 * 64
        out_ref[pl.ds(s, 64)] = (
            out_ref[pl.ds(s, 64)]
            + recv_ref[pl.ds(128, 64)].astype(jnp.float32)
        )

        send_buf_ref[pl.ds(0, 64)] = out_ref[pl.ds(s, 64)].astype(jnp.bfloat16)
        z_rdma = []
        for i, XR in enumerate((4, 8, 12)):
            z = pltpu.make_async_remote_copy(
                src_ref=send_buf_ref.at[pl.ds(0, 64)],
                dst_ref=recv_ref.at[pl.ds(192 + i * 64, 64)],
                send_sem=send_sems.at[2 + i],
                recv_sem=recv_sems.at[2 + i],
                device_id=(my ^ XR,),
                device_id_type=pl.DeviceIdType.MESH,
            )
            z.start()
            z_rdma.append(z)
        for z in z_rdma:
            z.wait_recv()
        out_ref[pl.ds(s, 64)] = (
            out_ref[pl.ds(s, 64)]
            + recv_ref[pl.ds(192, 64)].astype(jnp.float32)
            + recv_ref[pl.ds(256, 64)].astype(jnp.float32)
            + recv_ref[pl.ds(320, 64)].astype(jnp.float32)
        )
        for z in z_rdma:
            z.wait_send()

        send_buf_ref[pl.ds(0, 64)] = out_ref[pl.ds(s, 64)].astype(jnp.bfloat16)
        c_rdma = []
        for i, XR in enumerate((1, 2, 3)):
            c = pltpu.make_async_remote_copy(
                src_ref=send_buf_ref.at[pl.ds(0, 64)],
                dst_ref=recv_ref.at[pl.ds(384 + i * 64, 64)],
                send_sem=send_sems.at[5 + i],
                recv_sem=recv_sems.at[5 + i],
                device_id=(my ^ XR,),
                device_id_type=pl.DeviceIdType.MESH,
            )
            c.start()
            c_rdma.append(c)
        for i, XR in enumerate((1, 2, 3)):
            c_rdma[i].wait_recv()
            p = my ^ XR
            sp = ((p >> 1) & 1) * HALF + (p & 1) * 64
            out_ref[pl.ds(sp, 64)] = recv_ref[
                pl.ds(384 + i * 64, 64)].astype(jnp.float32)
        for c in c_rdma:
            c.wait_send()

    out = pl.pallas_call(
        body,
        out_shape=jax.ShapeDtypeStruct((ROWS, d_model), jnp.float32),
        in_specs=[pl.BlockSpec(memory_space=pltpu.VMEM)] * 5,
        out_specs=pl.BlockSpec(memory_space=pltpu.VMEM),
        scratch_shapes=[
            pltpu.VMEM((576, d_model), jnp.bfloat16),
            pltpu.VMEM((HALF, d_model), jnp.bfloat16),
            pltpu.SemaphoreType.DMA((8,)),
            pltpu.SemaphoreType.DMA((8,)),
        ],
        compiler_params=pltpu.CompilerParams(collective_id=0),
    )(x, Wq, Wo, K_ext, V_ext)
    return out.reshape(B, SQ, d_model)


# device time: 23702 ns/iter; 1.3366x vs baseline; 1.0304x over previous
import jax
import jax.numpy as jnp
from jax import lax
from jax.experimental import pallas as pl
from jax.experimental.pallas import tpu as pltpu

N_DEV = 16
B, SQ, D = 2, 128, 512
HQ_LOCAL, DH = 8, 64
ROWS = B * SQ
HALF = ROWS // 2


def kernel(x, Wq, Wo, K_ext, V_ext):
    d_model = Wo.shape[1]

    def body(x_ref, wq_ref, wo_ref, k_ref, v_ref, out_ref,
             recv_ref, send_buf_ref, send_sems, recv_sems):
        my = lax.axis_index("i")
        bit0 = my & 1
        bit1 = (my >> 1) & 1
        bf16 = jnp.bfloat16

        barrier_sem = pltpu.get_barrier_semaphore()
        for xr in (1, 2, 3, 4, 8, 12):
            pl.semaphore_signal(barrier_sem, inc=1,
                                device_id=(my ^ xr,),
                                device_id_type=pl.DeviceIdType.MESH)

        def compute_half(b):
            xb = x_ref[b]
            qb = jnp.dot(xb.astype(bf16), wq_ref[...].astype(bf16),
                         preferred_element_type=jnp.float32)
            q3 = qb.astype(bf16).reshape(SQ, HQ_LOCAL, DH)
            kv = k_ref[b].astype(bf16)
            vv = v_ref[b].astype(bf16)
            outs_h = []
            for h in range(HQ_LOCAL):
                sc = lax.dot_general(
                    q3[:, h, :], kv[:, h, :],
                    (((1,), (1,)), ((), ())),
                    preferred_element_type=jnp.float32) * 0.125
                sc = sc - jnp.max(sc, axis=-1, keepdims=True)
                p = jnp.exp(sc)
                l = jnp.sum(p, axis=-1, keepdims=True)
                oh = jnp.dot(p.astype(bf16), vv[:, h, :],
                             preferred_element_type=jnp.float32)
                outs_h.append(oh / l)
            attn_b = jnp.concatenate(outs_h, axis=-1)
            out_ref[pl.ds(b * HALF, HALF)] = jnp.dot(
                attn_b.astype(bf16), wo_ref[...].astype(bf16),
                preferred_element_type=jnp.float32)

        @pl.when(bit1 == 0)
        def _():
            compute_half(1)

        @pl.when(bit1 == 1)
        def _():
            compute_half(0)

        pl.semaphore_wait(barrier_sem, 5)

        send_start0 = (1 - bit1) * HALF
        send_buf_ref[pl.ds(0, HALF)] = out_ref[
            pl.ds(send_start0, HALF)].astype(jnp.bfloat16)
        rdma0 = pltpu.make_async_remote_copy(
            src_ref=send_buf_ref.at[pl.ds(0, HALF)],
            dst_ref=recv_ref.at[pl.ds(0, HALF)],
            send_sem=send_sems.at[0],
            recv_sem=recv_sems.at[0],
            device_id=(my ^ 3,),
            device_id_type=pl.DeviceIdType.MESH,
        )
        rdma0.start()

        @pl.when(bit1 == 0)
        def _():
            compute_half(0)

        @pl.when(bit1 == 1)
        def _():
            compute_half(1)

        rdma0.wait()
        s = bit1 * HALF
        out_ref[pl.ds(s, HALF)] = (
            out_ref[pl.ds(s, HALF)]
            + recv_ref[pl.ds(0, HALF)].astype(jnp.float32)
        )

        send_buf_ref[pl.ds(0, 64)] = out_ref[
            pl.ds(s + (1 - bit0) * 64, 64)].astype(jnp.bfloat16)
        rdma1 = pltpu.make_async_remote_copy(
            src_ref=send_buf_ref.at[pl.ds(0, 64)],
            dst_ref=recv_ref.at[pl.ds(128, 64)],
            send_sem=send_sems.at[1],
            recv_sem=recv_sems.at[1],
            device_id=(my ^ 1,),
            device_id_type=pl.DeviceIdType.MESH,
        )
        rdma1.start()
        rdma1.wait()
        s = s + bit0 * 64
        out_ref[pl.ds(s, 64)] = (
            out_ref[pl.ds(s, 64)]
            + recv_ref[pl.ds(128, 64)].astype(jnp.float32)
        )

        send_buf_ref[pl.ds(0, 64)] = out_ref[pl.ds(s, 64)].astype(jnp.bfloat16)
        z_rdma = []
        for i, XR in enumerate((4, 8, 12)):
            z = pltpu.make_async_remote_copy(
                src_ref=send_buf_ref.at[pl.ds(0, 64)],
                dst_ref=recv_ref.at[pl.ds(192 + i * 64, 64)],
                send_sem=send_sems.at[2 + i],
                recv_sem=recv_sems.at[2 + i],
                device_id=(my ^ XR,),
                device_id_type=pl.DeviceIdType.MESH,
            )
            z.start()
            z_rdma.append(z)
        for z in z_rdma:
            z.wait_recv()
        out_ref[pl.ds(s, 64)] = (
            out_ref[pl.ds(s, 64)]
            + recv_ref[pl.ds(192, 64)].astype(jnp.float32)
            + recv_ref[pl.ds(256, 64)].astype(jnp.float32)
            + recv_ref[pl.ds(320, 64)].astype(jnp.float32)
        )
        for z in z_rdma:
            z.wait_send()

        send_buf_ref[pl.ds(0, 64)] = out_ref[pl.ds(s, 64)].astype(jnp.bfloat16)
        c_rdma = []
        for i, XR in enumerate((1, 2, 3)):
            c = pltpu.make_async_remote_copy(
                src_ref=send_buf_ref.at[pl.ds(0, 64)],
                dst_ref=recv_ref.at[pl.ds(384 + i * 64, 64)],
                send_sem=send_sems.at[5 + i],
                recv_sem=recv_sems.at[5 + i],
                device_id=(my ^ XR,),
                device_id_type=pl.DeviceIdType.MESH,
            )
            c.start()
            c_rdma.append(c)
        for i, XR in enumerate((1, 2, 3)):
            c_rdma[i].wait_recv()
            p = my ^ XR
            sp = ((p >> 1) & 1) * HALF + (p & 1) * 64
            out_ref[pl.ds(sp, 64)] = recv_ref[
                pl.ds(384 + i * 64, 64)].astype(jnp.float32)
        for c in c_rdma:
            c.wait_send()

    out = pl.pallas_call(
        body,
        out_shape=jax.ShapeDtypeStruct((ROWS, d_model), jnp.float32),
        in_specs=[pl.BlockSpec(memory_space=pltpu.VMEM)] * 5,
        out_specs=pl.BlockSpec(memory_space=pltpu.VMEM),
        scratch_shapes=[
            pltpu.VMEM((576, d_model), jnp.bfloat16),
            pltpu.VMEM((HALF, d_model), jnp.bfloat16),
            pltpu.SemaphoreType.DMA((8,)),
            pltpu.SemaphoreType.DMA((8,)),
        ],
        compiler_params=pltpu.CompilerParams(collective_id=0),
    )(x, Wq, Wo, K_ext, V_ext)
    return out.reshape(B, SQ, d_model)


# device time: 23638 ns/iter; 1.3402x vs baseline; 1.0027x over previous
import jax
import jax.numpy as jnp
from jax import lax
from jax.experimental import pallas as pl
from jax.experimental.pallas import tpu as pltpu

N_DEV = 16
B, SQ, D = 2, 128, 512
HQ_LOCAL, DH = 8, 64
ROWS = B * SQ
HALF = ROWS // 2


def kernel(x, Wq, Wo, K_ext, V_ext):
    d_model = Wo.shape[1]

    def body(x_ref, wq_ref, wo_ref, k_ref, v_ref, out_ref,
             recv_ref, send_buf_ref, send_sems, recv_sems):
        my = lax.axis_index("i")
        bit0 = my & 1
        bit1 = (my >> 1) & 1
        bf16 = jnp.bfloat16

        barrier_sem = pltpu.get_barrier_semaphore()
        for xr in (1, 2, 3, 4, 8, 12):
            pl.semaphore_signal(barrier_sem, inc=1,
                                device_id=(my ^ xr,),
                                device_id_type=pl.DeviceIdType.MESH)

        def compute_half(b):
            xb = x_ref[b]
            qb = jnp.dot(xb.astype(bf16), wq_ref[...].astype(bf16),
                         preferred_element_type=jnp.float32)
            q3 = qb.astype(bf16).reshape(SQ, HQ_LOCAL, DH)
            kv = k_ref[b].astype(bf16)
            vv = v_ref[b].astype(bf16)
            outs_h = []
            for h in range(HQ_LOCAL):
                sc = lax.dot_general(
                    q3[:, h, :], kv[:, h, :],
                    (((1,), (1,)), ((), ())),
                    preferred_element_type=jnp.float32) * 0.125
                sc = sc - jnp.max(sc, axis=-1, keepdims=True)
                p = jnp.exp(sc)
                l = jnp.sum(p, axis=-1, keepdims=True)
                oh = jnp.dot(p.astype(bf16), vv[:, h, :],
                             preferred_element_type=jnp.float32)
                outs_h.append(oh / l)
            attn_b = jnp.concatenate(outs_h, axis=-1)
            out_ref[pl.ds(b * HALF, HALF)] = jnp.dot(
                attn_b.astype(bf16), wo_ref[...].astype(bf16),
                preferred_element_type=jnp.float32)

        @pl.when(bit1 == 0)
        def _():
            compute_half(1)

        @pl.when(bit1 == 1)
        def _():
            compute_half(0)

        pl.semaphore_wait(barrier_sem, 5)

        send_start0 = (1 - bit1) * HALF
        send_buf_ref[pl.ds(0, HALF)] = out_ref[
            pl.ds(send_start0, HALF)].astype(jnp.bfloat16)
        rdma0 = pltpu.make_async_remote_copy(
            src_ref=send_buf_ref.at[pl.ds(0, HALF)],
            dst_ref=recv_ref.at[pl.ds(0, HALF)],
            send_sem=send_sems.at[0],
            recv_sem=recv_sems.at[0],
            device_id=(my ^ 3,),
            device_id_type=pl.DeviceIdType.MESH,
        )
        rdma0.start()

        @pl.when(bit1 == 0)
        def _():
            compute_half(0)

        @pl.when(bit1 == 1)
        def _():
            compute_half(1)

        rdma0.wait()
        s = bit1 * HALF
        sq_send = s + (1 - bit0) * 64
        sq_keep = s + bit0 * 64
        fwd = (out_ref[pl.ds(sq_send, 64)]
               + recv_ref[pl.ds((1 - bit0) * 64, 64)].astype(jnp.float32))
        out_ref[pl.ds(sq_send, 64)] = fwd
        send_buf_ref[pl.ds(0, 64)] = fwd.astype(jnp.bfloat16)
        rdma1 = pltpu.make_async_remote_copy(
            src_ref=send_buf_ref.at[pl.ds(0, 64)],
            dst_ref=recv_ref.at[pl.ds(128, 64)],
            send_sem=send_sems.at[1],
            recv_sem=recv_sems.at[1],
            device_id=(my ^ 1,),
            device_id_type=pl.DeviceIdType.MESH,
        )
        rdma1.start()
        out_ref[pl.ds(sq_keep, 64)] = (
            out_ref[pl.ds(sq_keep, 64)]
            + recv_ref[pl.ds(bit0 * 64, 64)].astype(jnp.float32)
        )
        rdma1.wait()
        s = sq_keep
        seg = (out_ref[pl.ds(s, 64)]
               + recv_ref[pl.ds(128, 64)].astype(jnp.float32))
        out_ref[pl.ds(s, 64)] = seg

        send_buf_ref[pl.ds(0, 64)] = seg.astype(jnp.bfloat16)
        z_rdma = []
        for i, XR in enumerate((4, 8, 12)):
            z = pltpu.make_async_remote_copy(
                src_ref=send_buf_ref.at[pl.ds(0, 64)],
                dst_ref=recv_ref.at[pl.ds(192 + i * 64, 64)],
                send_sem=send_sems.at[2 + i],
                recv_sem=recv_sems.at[2 + i],
                device_id=(my ^ XR,),
                device_id_type=pl.DeviceIdType.MESH,
            )
            z.start()
            z_rdma.append(z)
        for z in z_rdma:
            z.wait_recv()
        seg2 = (out_ref[pl.ds(s, 64)]
                + recv_ref[pl.ds(192, 64)].astype(jnp.float32)
                + recv_ref[pl.ds(256, 64)].astype(jnp.float32)
                + recv_ref[pl.ds(320, 64)].astype(jnp.float32))
        out_ref[pl.ds(s, 64)] = seg2
        for z in z_rdma:
            z.wait_send()

        send_buf_ref[pl.ds(0, 64)] = seg2.astype(jnp.bfloat16)
        c_rdma = []
        for i, XR in enumerate((1, 2, 3)):
            c = pltpu.make_async_remote_copy(
                src_ref=send_buf_ref.at[pl.ds(0, 64)],
                dst_ref=recv_ref.at[pl.ds(384 + i * 64, 64)],
                send_sem=send_sems.at[5 + i],
                recv_sem=recv_sems.at[5 + i],
                device_id=(my ^ XR,),
                device_id_type=pl.DeviceIdType.MESH,
            )
            c.start()
            c_rdma.append(c)
        for i, XR in enumerate((1, 2, 3)):
            c_rdma[i].wait_recv()
            p = my ^ XR
            sp = ((p >> 1) & 1) * HALF + (p & 1) * 64
            out_ref[pl.ds(sp, 64)] = recv_ref[
                pl.ds(384 + i * 64, 64)].astype(jnp.float32)
        for c in c_rdma:
            c.wait_send()

    out = pl.pallas_call(
        body,
        out_shape=jax.ShapeDtypeStruct((ROWS, d_model), jnp.float32),
        in_specs=[pl.BlockSpec(memory_space=pltpu.VMEM)] * 5,
        out_specs=pl.BlockSpec(memory_space=pltpu.VMEM),
        scratch_shapes=[
            pltpu.VMEM((576, d_model), jnp.bfloat16),
            pltpu.VMEM((HALF, d_model), jnp.bfloat16),
            pltpu.SemaphoreType.DMA((8,)),
            pltpu.SemaphoreType.DMA((8,)),
        ],
        compiler_params=pltpu.CompilerParams(collective_id=0),
    )(x, Wq, Wo, K_ext, V_ext)
    return out.reshape(B, SQ, d_model)


# device time: 22825 ns/iter; 1.3879x vs baseline; 1.0356x over previous
import jax
import jax.numpy as jnp
from jax import lax
from jax.experimental import pallas as pl
from jax.experimental.pallas import tpu as pltpu

N_DEV = 16
B, SQ, D = 2, 128, 512
HQ_LOCAL, DH = 8, 64
ROWS = B * SQ
HALF = ROWS // 2


def kernel(x, Wq, Wo, K_ext, V_ext):
    d_model = Wo.shape[1]

    def body(x_ref, wq_ref, wo_ref, k_ref, v_ref, out_ref,
             recv_ref, send_buf_ref, send_sems, recv_sems):
        my = lax.axis_index("i")
        bit0 = my & 1
        bit1 = (my >> 1) & 1
        bf16 = jnp.bfloat16

        barrier_sem = pltpu.get_barrier_semaphore()
        for xr in (1, 2, 3, 4, 8, 12):
            pl.semaphore_signal(barrier_sem, inc=1,
                                device_id=(my ^ xr,),
                                device_id_type=pl.DeviceIdType.MESH)

        def compute_half(b):
            xb = x_ref[b]
            qb = jnp.dot(xb.astype(bf16), wq_ref[...].astype(bf16),
                         preferred_element_type=jnp.float32)
            q3 = qb.astype(bf16).reshape(SQ, HQ_LOCAL, DH)
            kv = k_ref[b].astype(bf16)
            vv = v_ref[b].astype(bf16)
            sc = lax.dot_general(
                q3, kv, (((2,), (2,)), ((1,), (1,))),
                preferred_element_type=jnp.float32) * 0.125
            sc = sc - jnp.max(sc, axis=-1, keepdims=True)
            p = jnp.exp(sc)
            l = jnp.sum(p, axis=-1, keepdims=True)
            o = lax.dot_general(
                p.astype(bf16), vv, (((2,), (0,)), ((0,), (1,))),
                preferred_element_type=jnp.float32)
            o = o / l
            attn_b = o.transpose(1, 0, 2).reshape(SQ, HQ_LOCAL * DH)
            out_ref[pl.ds(b * HALF, HALF)] = jnp.dot(
                attn_b.astype(bf16), wo_ref[...].astype(bf16),
                preferred_element_type=jnp.float32)

        @pl.when(bit1 == 0)
        def _():
            compute_half(1)

        @pl.when(bit1 == 1)
        def _():
            compute_half(0)

        pl.semaphore_wait(barrier_sem, 5)

        send_start0 = (1 - bit1) * HALF
        send_buf_ref[pl.ds(0, HALF)] = out_ref[
            pl.ds(send_start0, HALF)].astype(jnp.bfloat16)
        rdma0 = pltpu.make_async_remote_copy(
            src_ref=send_buf_ref.at[pl.ds(0, HALF)],
            dst_ref=recv_ref.at[pl.ds(0, HALF)],
            send_sem=send_sems.at[0],
            recv_sem=recv_sems.at[0],
            device_id=(my ^ 3,),
            device_id_type=pl.DeviceIdType.MESH,
        )
        rdma0.start()

        @pl.when(bit1 == 0)
        def _():
            compute_half(0)

        @pl.when(bit1 == 1)
        def _():
            compute_half(1)

        rdma0.wait()
        s = bit1 * HALF
        sq_send = s + (1 - bit0) * 64
        sq_keep = s + bit0 * 64
        fwd = (out_ref[pl.ds(sq_send, 64)]
               + recv_ref[pl.ds((1 - bit0) * 64, 64)].astype(jnp.float32))
        out_ref[pl.ds(sq_send, 64)] = fwd
        send_buf_ref[pl.ds(0, 64)] = fwd.astype(jnp.bfloat16)
        rdma1 = pltpu.make_async_remote_copy(
            src_ref=send_buf_ref.at[pl.ds(0, 64)],
            dst_ref=recv_ref.at[pl.ds(128, 64)],
            send_sem=send_sems.at[1],
            recv_sem=recv_sems.at[1],
            device_id=(my ^ 1,),
            device_id_type=pl.DeviceIdType.MESH,
        )
        rdma1.start()
        out_ref[pl.ds(sq_keep, 64)] = (
            out_ref[pl.ds(sq_keep, 64)]
            + recv_ref[pl.ds(bit0 * 64, 64)].astype(jnp.float32)
        )
        rdma1.wait()
        s = sq_keep
        seg = (out_ref[pl.ds(s, 64)]
               + recv_ref[pl.ds(128, 64)].astype(jnp.float32))
        out_ref[pl.ds(s, 64)] = seg

        send_buf_ref[pl.ds(0, 64)] = seg.astype(jnp.bfloat16)
        z_rdma = []
        for i, XR in enumerate((4, 8, 12)):
            z = pltpu.make_async_remote_copy(
                src_ref=send_buf_ref.at[pl.ds(0, 64)],
                dst_ref=recv_ref.at[pl.ds(192 + i * 64, 64)],
                send_sem=send_sems.at[2 + i],
                recv_sem=recv_sems.at[2 + i],
                device_id=(my ^ XR,),
                device_id_type=pl.DeviceIdType.MESH,
            )
            z.start()
            z_rdma.append(z)
        for z in z_rdma:
            z.wait_recv()
        seg2 = (out_ref[pl.ds(s, 64)]
                + recv_ref[pl.ds(192, 64)].astype(jnp.float32)
                + recv_ref[pl.ds(256, 64)].astype(jnp.float32)
                + recv_ref[pl.ds(320, 64)].astype(jnp.float32))
        out_ref[pl.ds(s, 64)] = seg2
        for z in z_rdma:
            z.wait_send()

        send_buf_ref[pl.ds(0, 64)] = seg2.astype(jnp.bfloat16)
        c_rdma = []
        for i, XR in enumerate((1, 2, 3)):
            c = pltpu.make_async_remote_copy(
                src_ref=send_buf_ref.at[pl.ds(0, 64)],
                dst_ref=recv_ref.at[pl.ds(384 + i * 64, 64)],
                send_sem=send_sems.at[5 + i],
                recv_sem=recv_sems.at[5 + i],
                device_id=(my ^ XR,),
                device_id_type=pl.DeviceIdType.MESH,
            )
            c.start()
            c_rdma.append(c)
        for i, XR in enumerate((1, 2, 3)):
            c_rdma[i].wait_recv()
            p = my ^ XR
            sp = ((p >> 1) & 1) * HALF + (p & 1) * 64
            out_ref[pl.ds(sp, 64)] = recv_ref[
                pl.ds(384 + i * 64, 64)].astype(jnp.float32)
        for c in c_rdma:
            c.wait_send()

    out = pl.pallas_call(
        body,
        out_shape=jax.ShapeDtypeStruct((ROWS, d_model), jnp.float32),
        in_specs=[pl.BlockSpec(memory_space=pltpu.VMEM)] * 5,
        out_specs=pl.BlockSpec(memory_space=pltpu.VMEM),
        scratch_shapes=[
            pltpu.VMEM((576, d_model), jnp.bfloat16),
            pltpu.VMEM((HALF, d_model), jnp.bfloat16),
            pltpu.SemaphoreType.DMA((8,)),
            pltpu.SemaphoreType.DMA((8,)),
        ],
        compiler_params=pltpu.CompilerParams(collective_id=0),
    )(x, Wq, Wo, K_ext, V_ext)
    return out.reshape(B, SQ, d_model)
